# baseline (speedup 1.0000x reference)
import jax
import jax.numpy as jnp
from jax.experimental import pallas as pl
from jax.experimental.pallas import tpu as pltpu
from jax.experimental.pallas import tpu_sc as plsc

D = 64
ROWS = 128
BR = 4
SCALE = 8.0


def kernel(x, table):
    B, L = x.shape
    N = B * L
    H = B // 2

    xi = x.T.reshape(L, 2, H).transpose(0, 2, 1).reshape(N // ROWS, ROWS)

    mesh = plsc.VectorSubcoreMesh(core_axis_name="c", subcore_axis_name="s")

    @pl.kernel(
        out_type=jax.ShapeDtypeStruct((N, D), jnp.float32),
        mesh=mesh,
        compiler_params=pltpu.CompilerParams(use_tc_tiling_on_sc=False),
    )
    def gather_k(table_hbm, i_hbm, o_hbm):
        def body(i_vmem, o_vmem):
            for r in range(BR):
                pltpu.sync_copy(table_hbm.at[i_vmem.at[r]],
                                o_vmem.at[pl.ds(r * ROWS, ROWS)])

        pltpu.emit_pipeline(
            body,
            grid=(N // ROWS // BR,),
            in_specs=[pl.BlockSpec((BR, ROWS), index_map=lambda i: (i, 0))],
            out_specs=[pl.BlockSpec((BR * ROWS, D), index_map=lambda i: (i, 0))],
            core_axis_name=("c", "s"),
            dimension_semantics=(pltpu.PARALLEL,),
        )(i_hbm, o_hbm)

    raw = gather_k(table, xi)
    pairs = raw.reshape(N // 2, 2 * D)

    CH = 1024

    def trans_k(p_ref, o_ref):
        for c in range(H // CH):
            v = p_ref[pl.ds(c * CH, CH), :] * SCALE
            o_ref[0, :, pl.ds(c * CH, CH)] = v[:, :D].T
            o_ref[0, :, pl.ds(H + c * CH, CH)] = v[:, D:].T

    out = pl.pallas_call(
        trans_k,
        grid=(L,),
        in_specs=[pl.BlockSpec((H, 2 * D), lambda l: (l, 0))],
        out_specs=pl.BlockSpec((1, D, B), lambda l: (l, 0, 0)),
        out_shape=jax.ShapeDtypeStruct((L, D, B), jnp.float32),
    )(pairs)

    return out.transpose(2, 0, 1)


# speedup vs baseline: 1.1785x; 1.1785x over previous
import jax
import jax.numpy as jnp
from jax.experimental import pallas as pl
from jax.experimental.pallas import tpu as pltpu
from jax.experimental.pallas import tpu_sc as plsc

D = 64
W = 256
SCALE = 8.0
CB = 16384
CC = 512
BB = 2048
CH = 512


def kernel(x, table):
    B, L = x.shape
    V = table.shape[0]
    N = B * L
    SPLIT = 31 * CB

    xt = x.T
    qi = jnp.where(xt < SPLIT, xt, xt - SPLIT).reshape(N // W, W)
    hb = (xt >= SPLIT).astype(jnp.int32).reshape(L, 1, B)

    def pack_k(lo_ref, hi_ref, o_ref):
        for c in range(CB // CC):
            sl = pl.ds(c * CC, CC)
            o_ref[sl, :D] = lo_ref[:, sl].T
            o_ref[sl, D:] = hi_ref[:, sl].T

    packed = pl.pallas_call(
        pack_k,
        grid=(SPLIT // CB,),
        in_specs=[
            pl.BlockSpec((D, CB), lambda i: (0, i)),
            pl.BlockSpec((D, CB), lambda i: (0, SPLIT // CB + i)),
        ],
        out_specs=pl.BlockSpec((CB, 2 * D), lambda i: (i, 0)),
        out_shape=jax.ShapeDtypeStruct((SPLIT, 2 * D), jnp.float32),
    )(table.T, table.T)

    mesh = plsc.VectorSubcoreMesh(core_axis_name="c", subcore_axis_name="s")

    @pl.kernel(
        out_type=jax.ShapeDtypeStruct((N, 2 * D), jnp.float32),
        mesh=mesh,
        compiler_params=pltpu.CompilerParams(use_tc_tiling_on_sc=False),
    )
    def gather_k(table_hbm, i_hbm, o_hbm):
        def body(i_vmem, o_vmem):
            pltpu.sync_copy(table_hbm.at[i_vmem.at[0]], o_vmem)

        pltpu.emit_pipeline(
            body,
            grid=(N // W,),
            in_specs=[pl.BlockSpec((1, W), index_map=lambda i: (i, 0))],
            out_specs=[pl.BlockSpec((W, 2 * D), index_map=lambda i: (i, 0))],
            core_axis_name=("c", "s"),
            dimension_semantics=(pltpu.PARALLEL,),
        )(i_hbm, o_hbm)

    raw = gather_k(packed, qi)

    def trans_k(p_ref, h_ref, o_ref):
        for c in range(BB // CH):
            sl = pl.ds(c * CH, CH)
            v = p_ref[sl, :]
            h = h_ref[0, :, sl]
            lo = v[:, :D].T
            hi = v[:, D:].T
            o_ref[0, :, sl] = jnp.where(h > 0, hi, lo) * SCALE

    out = pl.pallas_call(
        trans_k,
        grid=(L, B // BB),
        in_specs=[
            pl.BlockSpec((BB, 2 * D), lambda l, j: (l * (B // BB) + j, 0)),
            pl.BlockSpec((1, 1, BB), lambda l, j: (l, 0, j)),
        ],
        out_specs=pl.BlockSpec((1, D, BB), lambda l, j: (l, 0, j)),
        out_shape=jax.ShapeDtypeStruct((L, D, B), jnp.float32),
    )(raw, hb)

    return out.transpose(2, 0, 1)


# speedup vs baseline: 1.3103x; 1.1118x over previous
import jax
import jax.numpy as jnp
from jax.experimental import pallas as pl
from jax.experimental.pallas import tpu as pltpu
from jax.experimental.pallas import tpu_sc as plsc

D = 64
W = 256
SCALE = 8.0
CB = 16384
CC = 512
BB = 2048
CH = 512


def kernel(x, table):
    B, L = x.shape
    V = table.shape[0]
    N = B * L
    SPLIT = 31 * CB

    xt = x.T
    qi = jnp.where(xt < SPLIT, xt, xt - SPLIT).reshape(N // W, W)
    hb = (xt >= SPLIT).astype(jnp.int32).reshape(L, 1, B)

    def pack_k(lo_ref, hi_ref, o_ref):
        for c in range(CB // CC):
            sl = pl.ds(c * CC, CC)
            o_ref[sl, :D] = lo_ref[:, sl].T
            o_ref[sl, D:] = hi_ref[:, sl].T

    packed = pl.pallas_call(
        pack_k,
        grid=(SPLIT // CB,),
        in_specs=[
            pl.BlockSpec((D, CB), lambda i: (0, i)),
            pl.BlockSpec((D, CB), lambda i: (0, SPLIT // CB + i)),
        ],
        out_specs=pl.BlockSpec((CB, 2 * D), lambda i: (i, 0)),
        out_shape=jax.ShapeDtypeStruct((SPLIT, 2 * D), jnp.float32),
        compiler_params=pltpu.CompilerParams(
            dimension_semantics=("parallel",)),
    )(table.T, table.T)

    mesh = plsc.VectorSubcoreMesh(core_axis_name="c", subcore_axis_name="s")

    @pl.kernel(
        out_type=jax.ShapeDtypeStruct((N, 2 * D), jnp.float32),
        mesh=mesh,
        compiler_params=pltpu.CompilerParams(use_tc_tiling_on_sc=False),
    )
    def gather_k(table_hbm, i_hbm, o_hbm):
        def body(i_vmem, o_vmem):
            pltpu.sync_copy(table_hbm.at[i_vmem.at[0]], o_vmem)

        pltpu.emit_pipeline(
            body,
            grid=(N // W,),
            in_specs=[pl.BlockSpec((1, W), index_map=lambda i: (i, 0))],
            out_specs=[pl.BlockSpec((W, 2 * D), index_map=lambda i: (i, 0))],
            core_axis_name=("c", "s"),
            dimension_semantics=(pltpu.PARALLEL,),
        )(i_hbm, o_hbm)

    raw = gather_k(packed, qi)

    def trans_k(p_ref, h_ref, o_ref):
        for c in range(BB // CH):
            sl = pl.ds(c * CH, CH)
            t = (p_ref[sl, :] * SCALE).T
            h = h_ref[0, :, sl]
            o_ref[0, :, sl] = jnp.where(h > 0, t[D:], t[:D])

    out = pl.pallas_call(
        trans_k,
        grid=(L, B // BB),
        in_specs=[
            pl.BlockSpec((BB, 2 * D), lambda l, j: (l * (B // BB) + j, 0)),
            pl.BlockSpec((1, 1, BB), lambda l, j: (l, 0, j)),
        ],
        out_specs=pl.BlockSpec((1, D, BB), lambda l, j: (l, 0, j)),
        out_shape=jax.ShapeDtypeStruct((L, D, B), jnp.float32),
        compiler_params=pltpu.CompilerParams(
            dimension_semantics=("parallel", "parallel")),
    )(raw, hb)

    return out.transpose(2, 0, 1)


# speedup vs baseline: 1.4781x; 1.1281x over previous
import jax
import jax.numpy as jnp
from jax.experimental import pallas as pl
from jax.experimental.pallas import tpu as pltpu
from jax.experimental.pallas import tpu_sc as plsc

D = 64
W = 400
SCALE = 8.0
CB = 16384
CC = 512
BB = 4096
CH = 512


def kernel(x, table):
    B, L = x.shape
    V = table.shape[0]
    N = B * L
    SPLIT = 31 * CB

    xt = x.T
    qi = jnp.where(xt < SPLIT, xt, xt - SPLIT).reshape(N // W, W)
    hb = (xt >= SPLIT).astype(jnp.int32).reshape(L, 1, B)

    def pack_k(lo_ref, hi_ref, o_ref):
        for c in range(CB // CC):
            sl = pl.ds(c * CC, CC)
            o_ref[sl, :D] = lo_ref[:, sl].T
            o_ref[sl, D:] = hi_ref[:, sl].T

    packed = pl.pallas_call(
        pack_k,
        grid=(SPLIT // CB,),
        in_specs=[
            pl.BlockSpec((D, CB), lambda i: (0, i)),
            pl.BlockSpec((D, CB), lambda i: (0, SPLIT // CB + i)),
        ],
        out_specs=pl.BlockSpec((CB, 2 * D), lambda i: (i, 0)),
        out_shape=jax.ShapeDtypeStruct((SPLIT, 2 * D), jnp.float32),
        compiler_params=pltpu.CompilerParams(
            dimension_semantics=("parallel",)),
    )(table.T, table.T)

    mesh = plsc.VectorSubcoreMesh(core_axis_name="c", subcore_axis_name="s")

    @pl.kernel(
        out_type=jax.ShapeDtypeStruct((N, 2 * D), jnp.float32),
        mesh=mesh,
        compiler_params=pltpu.CompilerParams(use_tc_tiling_on_sc=False),
    )
    def gather_k(table_hbm, i_hbm, o_hbm):
        def body(i_vmem, o_vmem):
            pltpu.sync_copy(table_hbm.at[i_vmem.at[0]], o_vmem)

        pltpu.emit_pipeline(
            body,
            grid=(N // W,),
            in_specs=[pl.BlockSpec((1, W), index_map=lambda i: (i, 0))],
            out_specs=[pl.BlockSpec((W, 2 * D), index_map=lambda i: (i, 0))],
            core_axis_name=("c", "s"),
            dimension_semantics=(pltpu.PARALLEL,),
        )(i_hbm, o_hbm)

    raw = gather_k(packed, qi)

    def trans_k(p_ref, h_ref, o_ref):
        for c in range(BB // CH):
            sl = pl.ds(c * CH, CH)
            t = (p_ref[sl, :] * SCALE).T
            h = h_ref[0, :, sl]
            o_ref[0, :, sl] = jnp.where(h > 0, t[D:], t[:D])

    out = pl.pallas_call(
        trans_k,
        grid=(L, B // BB),
        in_specs=[
            pl.BlockSpec((BB, 2 * D), lambda l, j: (l * (B // BB) + j, 0)),
            pl.BlockSpec((1, 1, BB), lambda l, j: (l, 0, j)),
        ],
        out_specs=pl.BlockSpec((1, D, BB), lambda l, j: (l, 0, j)),
        out_shape=jax.ShapeDtypeStruct((L, D, B), jnp.float32),
        compiler_params=pltpu.CompilerParams(
            dimension_semantics=("parallel", "parallel")),
    )(raw, hb)

    return out.transpose(2, 0, 1)


# speedup vs baseline: 1.5827x; 1.0708x over previous
import jax
import jax.numpy as jnp
from jax.experimental import pallas as pl
from jax.experimental.pallas import tpu as pltpu
from jax.experimental.pallas import tpu_sc as plsc

D = 64
W = 320
SCALE = 8.0
CB = 16384
CC = 512
BB = 4096
CH = 512
NC = 5


def kernel(x, table):
    B, L = x.shape
    V = table.shape[0]
    N = B * L
    LC = L // NC
    NL = LC * B
    SPLIT = 31 * CB

    xt = x.T
    qi = jnp.where(xt < SPLIT, xt, xt - SPLIT)
    hb = (xt >= SPLIT).astype(jnp.int32).reshape(L, 1, B)

    def pack_k(lo_ref, hi_ref, o_ref):
        for c in range(CB // CC):
            sl = pl.ds(c * CC, CC)
            o_ref[sl, :D] = lo_ref[:, sl].T
            o_ref[sl, D:] = hi_ref[:, sl].T

    packed = pl.pallas_call(
        pack_k,
        grid=(SPLIT // CB,),
        in_specs=[
            pl.BlockSpec((D, CB), lambda i: (0, i)),
            pl.BlockSpec((D, CB), lambda i: (0, SPLIT // CB + i)),
        ],
        out_specs=pl.BlockSpec((CB, 2 * D), lambda i: (i, 0)),
        out_shape=jax.ShapeDtypeStruct((SPLIT, 2 * D), jnp.float32),
        compiler_params=pltpu.CompilerParams(
            dimension_semantics=("parallel",)),
    )(table.T, table.T)

    mesh = plsc.VectorSubcoreMesh(core_axis_name="c", subcore_axis_name="s")

    def gather_chunk(idx_chunk):
        @pl.kernel(
            out_type=jax.ShapeDtypeStruct((NL, 2 * D), jnp.float32),
            mesh=mesh,
            compiler_params=pltpu.CompilerParams(use_tc_tiling_on_sc=False),
        )
        def gather_k(table_hbm, i_hbm, o_hbm):
            def body(i_vmem, o_vmem):
                pltpu.sync_copy(table_hbm.at[i_vmem.at[0]], o_vmem)

            pltpu.emit_pipeline(
                body,
                grid=(NL // W,),
                in_specs=[pl.BlockSpec((1, W), index_map=lambda i: (i, 0))],
                out_specs=[pl.BlockSpec((W, 2 * D),
                                        index_map=lambda i: (i, 0))],
                core_axis_name=("c", "s"),
                dimension_semantics=(pltpu.PARALLEL,),
            )(i_hbm, o_hbm)

        return gather_k(packed, idx_chunk)

    raws = [gather_chunk(qi[c * LC:(c + 1) * LC].reshape(NL // W, W))
            for c in range(NC)]

    def trans_first(p_ref, h_ref, o_ref):
        _trans_body(p_ref, h_ref, o_ref)

    def trans_next(prev_ref, p_ref, h_ref, o_ref):
        del prev_ref
        _trans_body(p_ref, h_ref, o_ref)

    def _trans_body(p_ref, h_ref, o_ref):
        for c in range(BB // CH):
            sl = pl.ds(c * CH, CH)
            t = (p_ref[sl, :] * SCALE).T
            h = h_ref[0, :, sl]
            o_ref[0, :, sl] = jnp.where(h > 0, t[D:], t[:D])

    JB = B // BB
    out = None
    for c in range(NC):
        p_spec = pl.BlockSpec((BB, 2 * D), lambda l, j: (l * JB + j, 0))
        h_spec = pl.BlockSpec((1, 1, BB),
                              lambda l, j, c=c: (c * LC + l, 0, j))
        o_spec = pl.BlockSpec((1, D, BB),
                              lambda l, j, c=c: (c * LC + l, 0, j))
        if out is None:
            out = pl.pallas_call(
                trans_first,
                grid=(LC, JB),
                in_specs=[p_spec, h_spec],
                out_specs=o_spec,
                out_shape=jax.ShapeDtypeStruct((L, D, B), jnp.float32),
                compiler_params=pltpu.CompilerParams(
                    dimension_semantics=("parallel", "parallel")),
            )(raws[c], hb)
        else:
            out = pl.pallas_call(
                trans_next,
                grid=(LC, JB),
                in_specs=[pl.BlockSpec(memory_space=pl.ANY),
                          p_spec, h_spec],
                out_specs=o_spec,
                out_shape=jax.ShapeDtypeStruct((L, D, B), jnp.float32),
                input_output_aliases={0: 0},
                compiler_params=pltpu.CompilerParams(
                    dimension_semantics=("parallel", "parallel")),
            )(out, raws[c], hb)

    return out.transpose(2, 0, 1)
